# baseline (speedup 1.0000x reference)
import functools

import jax
import jax.numpy as jnp
from jax import lax
from jax.experimental import pallas as pl
from jax.experimental.pallas import tpu as pltpu
from jax.experimental.pallas import tpu_sc as plsc

D = 64
L = 16
IC = 128


def _fm_sc(uidx, iidx, ut_t, it_t, params):
    B = uidx.shape[0]
    info = plsc.get_sparse_core_info()
    NC, NS = info.num_cores, info.num_subcores
    NW = NC * NS
    b_per_w = B // NW
    n_chunks = b_per_w // IC

    mesh = plsc.VectorSubcoreMesh(core_axis_name="c", subcore_axis_name="s")

    @functools.partial(
        pl.kernel,
        mesh=mesh,
        out_type=jax.ShapeDtypeStruct((B,), jnp.float32),
        compiler_params=pltpu.CompilerParams(
            needs_layout_passes=False, use_tc_tiling_on_sc=False),
        scratch_types=[
            pltpu.VMEM((b_per_w,), jnp.int32),
            pltpu.VMEM((b_per_w,), jnp.int32),
            pltpu.VMEM((b_per_w,), jnp.float32),
            pltpu.VMEM((b_per_w,), jnp.float32),
            pltpu.VMEM((b_per_w,), jnp.float32),
            pltpu.VMEM((b_per_w,), jnp.float32),
            pltpu.VMEM((params.shape[0],), jnp.float32),
            pltpu.VMEM((b_per_w,), jnp.float32),
            pltpu.SemaphoreType.DMA,
            pltpu.SemaphoreType.DMA,
            pltpu.SemaphoreType.DMA,
            pltpu.SemaphoreType.DMA,
        ],
    )
    def k(uidx_hbm, iidx_hbm, ut_hbm, it_hbm, p_hbm, out_hbm,
          uidx_v, iidx_v, ub0, ub1, ib0, ib1, w_v, acc_v,
          semu0, semu1, semi0, semi1):
        wid = lax.axis_index("s") * NC + lax.axis_index("c")
        base = wid * b_per_w
        pltpu.sync_copy(uidx_hbm.at[pl.ds(base, b_per_w)], uidx_v)
        pltpu.sync_copy(iidx_hbm.at[pl.ds(base, b_per_w)], iidx_v)
        pltpu.sync_copy(p_hbm, w_v)

        ubufs, ibufs = (ub0, ub1), (ib0, ib1)
        usems, isems = (semu0, semu1), (semi0, semi1)

        def copies(d, s):
            cps = []
            for c in range(n_chunks):
                sl = pl.ds(c * IC, IC)
                cps.append(pltpu.make_async_copy(
                    ut_hbm.at[d].at[uidx_v.at[sl]], ubufs[s].at[sl],
                    usems[s]))
                cps.append(pltpu.make_async_copy(
                    it_hbm.at[d].at[iidx_v.at[sl]], ibufs[s].at[sl],
                    isems[s]))
            return cps

        def fire(d, s):
            for cp in copies(d, s):
                cp.start()

        def drain(d, s):
            for cp in copies(d, s):
                cp.wait()

        bias = w_v[pl.ds(D * L, L)]

        def initg(g, carry):
            acc_v[pl.ds(g * L, L)] = bias
            return carry

        lax.fori_loop(0, b_per_w // L, initg, 0)

        def plane(d, s):
            wv = w_v[pl.ds(d * L, L)]
            ubuf, ibuf = ubufs[s], ibufs[s]

            def g(gi, carry):
                sl = pl.ds(gi * L, L)
                uu = ubuf[sl]
                ii = ibuf[sl]
                acc_v[sl] = acc_v[sl] + uu * (ii + wv) + ii * wv
                return carry

            lax.fori_loop(0, b_per_w // L, g, 0)

        fire(0, 0)
        fire(1, 1)

        def body(t, carry):
            d0 = 2 * t
            drain(d0, 0)
            plane(d0, 0)

            @pl.when(t < D // 2 - 1)
            def _():
                fire(d0 + 2, 0)

            drain(d0 + 1, 1)
            plane(d0 + 1, 1)

            @pl.when(t < D // 2 - 1)
            def _():
                fire(d0 + 3, 1)

            return carry

        lax.fori_loop(0, D // 2, body, 0)

        pltpu.sync_copy(acc_v, out_hbm.at[pl.ds(base, b_per_w)])

    return k(uidx, iidx, ut_t, it_t, params)


def kernel(user_idx, item_idx, user_table, item_table, w, b):
    ui = user_idx.astype(jnp.int32)
    ii = item_idx.astype(jnp.int32)
    ut_t = user_table.T
    it_t = item_table.T
    params = jnp.concatenate(
        [jnp.repeat(w.astype(jnp.float32), L),
         jnp.broadcast_to(b.astype(jnp.float32), (L,)),
         jnp.zeros((112,), jnp.float32)])
    return _fm_sc(ui, ii, ut_t, it_t, params)


# speedup vs baseline: 7.7060x; 7.7060x over previous
import functools

import jax
import jax.numpy as jnp
from jax import lax
from jax.experimental import pallas as pl
from jax.experimental.pallas import tpu as pltpu
from jax.experimental.pallas import tpu_sc as plsc

D = 64
L = 16
IC = 128


def _fm_sc(uidx, iidx, ut, it, params):
    B = uidx.shape[0]
    info = plsc.get_sparse_core_info()
    NC, NS = info.num_cores, info.num_subcores
    NW = NC * NS
    b_per_w = B // NW
    n_chunks = b_per_w // IC

    mesh = plsc.VectorSubcoreMesh(core_axis_name="c", subcore_axis_name="s")

    @functools.partial(
        pl.kernel,
        mesh=mesh,
        out_type=jax.ShapeDtypeStruct((B,), jnp.float32),
        compiler_params=pltpu.CompilerParams(
            needs_layout_passes=False, use_tc_tiling_on_sc=False),
        scratch_types=[
            pltpu.VMEM((b_per_w,), jnp.int32),
            pltpu.VMEM((b_per_w,), jnp.int32),
            pltpu.VMEM((b_per_w, D), jnp.float32),
            pltpu.VMEM((b_per_w, D), jnp.float32),
            pltpu.VMEM((params.shape[0],), jnp.float32),
            pltpu.VMEM((b_per_w,), jnp.float32),
            pltpu.SemaphoreType.DMA,
        ],
    )
    def k(uidx_hbm, iidx_hbm, ut_hbm, it_hbm, p_hbm, out_hbm,
          uidx_v, iidx_v, urows, irows, w_v, out_v, sem):
        wid = lax.axis_index("s") * NC + lax.axis_index("c")
        base = wid * b_per_w
        pltpu.sync_copy(uidx_hbm.at[pl.ds(base, b_per_w)], uidx_v)
        pltpu.sync_copy(iidx_hbm.at[pl.ds(base, b_per_w)], iidx_v)
        pltpu.sync_copy(p_hbm, w_v)

        def copies():
            cps = []
            for c in range(n_chunks):
                sl = pl.ds(c * IC, IC)
                cps.append(pltpu.make_async_copy(
                    ut_hbm.at[uidx_v.at[sl]], urows.at[sl], sem))
                cps.append(pltpu.make_async_copy(
                    it_hbm.at[iidx_v.at[sl]], irows.at[sl], sem))
            return cps

        for cp in copies():
            cp.start()
        for cp in copies():
            cp.wait()

        w0 = w_v[pl.ds(0, L)]
        w1 = w_v[pl.ds(L, L)]
        w2 = w_v[pl.ds(2 * L, L)]
        w3 = w_v[pl.ds(3 * L, L)]
        bias = w_v[pl.ds(D, L)]

        iota = lax.iota(jnp.int32, L)
        perms = [iota ^ sh for sh in (8, 4, 2, 1)]
        lane0 = iota == 0

        def row(e, carry):
            u0 = urows[e, pl.ds(0, L)]
            u1 = urows[e, pl.ds(L, L)]
            u2 = urows[e, pl.ds(2 * L, L)]
            u3 = urows[e, pl.ds(3 * L, L)]
            i0 = irows[e, pl.ds(0, L)]
            i1 = irows[e, pl.ds(L, L)]
            i2 = irows[e, pl.ds(2 * L, L)]
            i3 = irows[e, pl.ds(3 * L, L)]
            acc = (u0 * (i0 + w0) + i0 * w0
                   + u1 * (i1 + w1) + i1 * w1
                   + u2 * (i2 + w2) + i2 * w2
                   + u3 * (i3 + w3) + i3 * w3)
            for p in perms:
                acc = acc + jnp.take_along_axis(acc, p, axis=0)
            acc = acc + bias
            plsc.store_scatter(out_v, [jnp.full((L,), e, jnp.int32)],
                               acc, mask=lane0)
            return carry

        lax.fori_loop(0, b_per_w, row, 0)

        pltpu.sync_copy(out_v, out_hbm.at[pl.ds(base, b_per_w)])

    return k(uidx, iidx, ut, it, params)


def kernel(user_idx, item_idx, user_table, item_table, w, b):
    ui = user_idx.astype(jnp.int32)
    ii = item_idx.astype(jnp.int32)
    params = jnp.concatenate(
        [w.astype(jnp.float32),
         jnp.broadcast_to(b.astype(jnp.float32), (L,)),
         jnp.zeros((128 - D - L,), jnp.float32)])
    return _fm_sc(ui, ii, user_table, item_table, params)


# speedup vs baseline: 7.8104x; 1.0136x over previous
import functools

import jax
import jax.numpy as jnp
from jax import lax
from jax.experimental import pallas as pl
from jax.experimental.pallas import tpu as pltpu
from jax.experimental.pallas import tpu_sc as plsc

D = 64
L = 16
IC = 128


def _fm_sc(uidx, iidx, ut, it, params):
    B = uidx.shape[0]
    info = plsc.get_sparse_core_info()
    NC, NS = info.num_cores, info.num_subcores
    NW = NC * NS
    b_per_w = B // NW
    n_chunks = b_per_w // IC

    mesh = plsc.VectorSubcoreMesh(core_axis_name="c", subcore_axis_name="s")

    @functools.partial(
        pl.kernel,
        mesh=mesh,
        out_type=jax.ShapeDtypeStruct((B,), jnp.float32),
        compiler_params=pltpu.CompilerParams(
            needs_layout_passes=False, use_tc_tiling_on_sc=False),
        scratch_types=[
            pltpu.VMEM((b_per_w,), jnp.int32),
            pltpu.VMEM((b_per_w,), jnp.int32),
            pltpu.VMEM((b_per_w, D), jnp.float32),
            pltpu.VMEM((b_per_w, D), jnp.float32),
            pltpu.VMEM((params.shape[0],), jnp.float32),
            pltpu.VMEM((b_per_w,), jnp.float32),
            pltpu.SemaphoreType.DMA,
        ],
    )
    def k(uidx_hbm, iidx_hbm, ut_hbm, it_hbm, p_hbm, out_hbm,
          uidx_v, iidx_v, urows, irows, w_v, out_v, sem):
        wid = lax.axis_index("s") * NC + lax.axis_index("c")
        base = wid * b_per_w
        pltpu.sync_copy(uidx_hbm.at[pl.ds(base, b_per_w)], uidx_v)
        pltpu.sync_copy(iidx_hbm.at[pl.ds(base, b_per_w)], iidx_v)
        pltpu.sync_copy(p_hbm, w_v)

        def copies():
            cps = []
            for c in range(n_chunks):
                sl = pl.ds(c * IC, IC)
                cps.append(pltpu.make_async_copy(
                    ut_hbm.at[uidx_v.at[sl]], urows.at[sl], sem))
                cps.append(pltpu.make_async_copy(
                    it_hbm.at[iidx_v.at[sl]], irows.at[sl], sem))
            return cps

        for cp in copies():
            cp.start()
        for cp in copies():
            cp.wait()

        ws = [w_v[pl.ds(s * L, L)] for s in range(D // L)]
        bias = w_v[pl.ds(D, L)]

        iota = lax.iota(jnp.int32, L)
        perms = [iota ^ (1 << k) for k in range(4)]
        masks = [(iota & (1 << k)) == 0 for k in range(4)]

        def block(blk, carry):
            base16 = blk * L
            vs = []
            for j in range(L):
                e = base16 + j
                acc = None
                for s in range(D // L):
                    t = ((urows[e, pl.ds(s * L, L)] + ws[s])
                         * (irows[e, pl.ds(s * L, L)] + ws[s]))
                    acc = t if acc is None else acc + t
                vs.append(acc)
            for k in range(4):
                m, p = masks[k], perms[k]
                vs = [jnp.where(m, a, b) +
                      jnp.take_along_axis(jnp.where(m, b, a), p, axis=0)
                      for a, b in zip(vs[0::2], vs[1::2])]
            out_v[pl.ds(base16, L)] = vs[0] + bias
            return carry

        lax.fori_loop(0, b_per_w // L, block, 0)

        pltpu.sync_copy(out_v, out_hbm.at[pl.ds(base, b_per_w)])

    return k(uidx, iidx, ut, it, params)


def kernel(user_idx, item_idx, user_table, item_table, w, b):
    ui = user_idx.astype(jnp.int32)
    ii = item_idx.astype(jnp.int32)
    wf = w.astype(jnp.float32)
    eff_bias = b.astype(jnp.float32) - jnp.sum(wf * wf)
    params = jnp.concatenate(
        [wf,
         jnp.broadcast_to(eff_bias, (L,)),
         jnp.zeros((128 - D - L,), jnp.float32)])
    return _fm_sc(ui, ii, user_table, item_table, params)


# speedup vs baseline: 7.8119x; 1.0002x over previous
import functools

import jax
import jax.numpy as jnp
from jax import lax
from jax.experimental import pallas as pl
from jax.experimental.pallas import tpu as pltpu
from jax.experimental.pallas import tpu_sc as plsc

D = 64
L = 16
IC = 128


def _fm_sc(uidx, iidx, ut, it, params):
    B = uidx.shape[0]
    info = plsc.get_sparse_core_info()
    NC, NS = info.num_cores, info.num_subcores
    NW = NC * NS
    b_per_w = B // NW
    n_chunks = b_per_w // IC

    mesh = plsc.VectorSubcoreMesh(core_axis_name="c", subcore_axis_name="s")

    @functools.partial(
        pl.kernel,
        mesh=mesh,
        out_type=jax.ShapeDtypeStruct((B,), jnp.float32),
        compiler_params=pltpu.CompilerParams(
            needs_layout_passes=False,
            use_tc_tiling_on_sc=False),
        scratch_types=[
            pltpu.VMEM((b_per_w,), jnp.int32),
            pltpu.VMEM((b_per_w,), jnp.int32),
            pltpu.VMEM((b_per_w, D), jnp.float32),
            pltpu.VMEM((b_per_w, D), jnp.float32),
            pltpu.VMEM((params.shape[0],), jnp.float32),
            pltpu.VMEM((b_per_w,), jnp.float32),
            pltpu.SemaphoreType.DMA,
        ],
    )
    def k(uidx_hbm, iidx_hbm, ut_hbm, it_hbm, p_hbm, out_hbm,
          uidx_v, iidx_v, urows, irows, w_v, out_v, sem):
        wid = lax.axis_index("s") * NC + lax.axis_index("c")
        base = wid * b_per_w
        if True:
            pltpu.sync_copy(uidx_hbm.at[pl.ds(base, b_per_w)], uidx_v)
            pltpu.sync_copy(iidx_hbm.at[pl.ds(base, b_per_w)], iidx_v)
            pltpu.sync_copy(p_hbm, w_v)

        def copies():
            cps = []
            for c in range(n_chunks):
                sl = pl.ds(c * IC, IC)
                cps.append(pltpu.make_async_copy(
                    ut_hbm.at[uidx_v.at[sl]], urows.at[sl], sem))
                cps.append(pltpu.make_async_copy(
                    it_hbm.at[iidx_v.at[sl]], irows.at[sl], sem))
            return cps

        if True:
            for cp in copies():
                cp.start()
            for cp in copies():
                cp.wait()

        ws = [w_v[pl.ds(s * L, L)] for s in range(D // L)]
        bias = w_v[pl.ds(D, L)]

        iota = lax.iota(jnp.int32, L)
        perms = [iota ^ (1 << k) for k in range(4)]
        masks = [(iota & (1 << k)) == 0 for k in range(4)]

        def block(blk, carry):
            base16 = blk * L
            vs = []
            for j in range(L):
                e = base16 + j
                acc = None
                for s in range(D // L):
                    t = ((urows[e, pl.ds(s * L, L)] + ws[s])
                         * (irows[e, pl.ds(s * L, L)] + ws[s]))
                    acc = t if acc is None else acc + t
                vs.append(acc)
            for k in range(4):
                m, p = masks[k], perms[k]
                vs = [jnp.where(m, a, b) +
                      jnp.take_along_axis(jnp.where(m, b, a), p, axis=0)
                      for a, b in zip(vs[0::2], vs[1::2])]
            out_v[pl.ds(base16, L)] = vs[0] + bias
            return carry

        lax.fori_loop(0, b_per_w // L, block, 0)

        pltpu.sync_copy(out_v, out_hbm.at[pl.ds(base, b_per_w)])

    return k(uidx, iidx, ut, it, params)


def kernel(user_idx, item_idx, user_table, item_table, w, b):
    ui = user_idx.astype(jnp.int32)
    ii = item_idx.astype(jnp.int32)
    wf = w.astype(jnp.float32)
    eff_bias = b.astype(jnp.float32) - jnp.sum(wf * wf)
    params = jnp.concatenate(
        [wf,
         jnp.broadcast_to(eff_bias, (L,)),
         jnp.zeros((128 - D - L,), jnp.float32)])
    return _fm_sc(ui, ii, user_table, item_table, params)
